# baseline (speedup 1.0000x reference)
import functools

import jax
import jax.numpy as jnp
from jax import lax
from jax.experimental import pallas as pl
from jax.experimental.pallas import tpu as pltpu
from jax.experimental.pallas import tpu_sc as plsc

N = 10000
E = 320000
D = 128
C = 10
G = 128

NC = 2
NS = 16
NW = NC * NS

K = 64
EPT = E // NW
CH = EPT // K
KT = EPT - CH * K
NPAD = 10240
ZR = 64
DEPTH = 4


def _scseg(h, src_e, dst_e):
    mesh = plsc.VectorSubcoreMesh(core_axis_name="c", subcore_axis_name="s")

    @functools.partial(
        pl.kernel,
        mesh=mesh,
        out_type=jax.ShapeDtypeStruct((2 * N, D), jnp.float32),
        scratch_types=(
            [pltpu.VMEM((EPT,), jnp.int32)]
            + [pltpu.VMEM((K,), jnp.int32)] * DEPTH
            + [pltpu.VMEM((KT,), jnp.int32)]
            + [pltpu.VMEM((K, D), jnp.float32)] * DEPTH
            + [pltpu.VMEM_SHARED((NPAD, D), jnp.float32)]
            + [pltpu.SemaphoreType.DMA] * (2 * DEPTH)
        ),
    )
    def k(h_hbm, src_hbm, dst_hbm, out_hbm, src_all, *rest):
        dst_vs = rest[0:DEPTH]
        dst_vt = rest[DEPTH]
        rows_vs = rest[DEPTH + 1:2 * DEPTH + 1]
        acc_sh = rest[2 * DEPTH + 1]
        sgs = rest[2 * DEPTH + 2:3 * DEPTH + 2]
        sds = rest[3 * DEPTH + 2:4 * DEPTH + 2]
        c = lax.axis_index("c")
        s = lax.axis_index("s")

        def zrow(i, carry):
            for j in range(D // 16):
                rows_vs[0][i, pl.ds(j * 16, 16)] = jnp.zeros((16,),
                                                             jnp.float32)
            return carry

        lax.fori_loop(0, ZR, zrow, 0)

        def zcopy(i, carry):
            pltpu.sync_copy(rows_vs[0], acc_sh.at[pl.ds(s * 640 + i * ZR, ZR)])
            return carry

        lax.fori_loop(0, 640 // ZR, zcopy, 0)
        plsc.subcore_barrier()

        wid = c * NS + s
        ebase = wid * EPT
        pltpu.sync_copy(src_hbm.at[pl.ds(ebase, EPT)], src_all)

        def gather(ch, rows, sem):
            return pltpu.async_copy(
                h_hbm.at[src_all.at[pl.ds(ch * K, K)]], rows, sem)

        def gwait(ch, rows, sem):
            pltpu.make_async_copy(
                h_hbm.at[src_all.at[pl.ds(ch * K, K)]], rows, sem).wait()

        def dstage(ch, dst_v, sem):
            pltpu.async_copy(dst_hbm.at[pl.ds(ebase + ch * K, K)], dst_v, sem)

        def dwait(ch, dst_v, sem):
            pltpu.make_async_copy(
                dst_hbm.at[pl.ds(ebase + ch * K, K)], dst_v, sem).wait()

        def scat(rows, dst_v):
            pltpu.sync_copy(rows, acc_sh.at[dst_v], add=True)

        pltpu.sync_copy(dst_hbm.at[pl.ds(ebase + CH * K, KT)], dst_vt)
        pltpu.async_copy(
            h_hbm.at[src_all.at[pl.ds(CH * K, KT)]],
            rows_vs[0].at[pl.ds(0, KT)], sgs[0]).wait()
        pltpu.sync_copy(rows_vs[0].at[pl.ds(0, KT)], acc_sh.at[dst_vt],
                        add=True)

        for u in range(DEPTH):
            gather(u, rows_vs[u], sgs[u])
            dstage(u, dst_vs[u], sds[u])

        def body(t, carry):
            for u in range(DEPTH):
                ch = DEPTH * t + u
                gwait(ch, rows_vs[u], sgs[u])
                dwait(ch, dst_vs[u], sds[u])
                scat(rows_vs[u], dst_vs[u])

                @pl.when(ch + DEPTH < CH)
                def _():
                    gather(ch + DEPTH, rows_vs[u], sgs[u])
                    dstage(ch + DEPTH, dst_vs[u], sds[u])

            return carry

        lax.fori_loop(0, CH // DEPTH, body, 0)
        plsc.subcore_barrier()

        @pl.when(s < NS - 1)
        def _():
            pltpu.sync_copy(acc_sh.at[pl.ds(s * 640, 640)],
                            out_hbm.at[pl.ds(c * N + s * 640, 640)])

        @pl.when(s == NS - 1)
        def _():
            pltpu.sync_copy(acc_sh.at[pl.ds(9600, N - 9600)],
                            out_hbm.at[pl.ds(c * N + 9600, N - 9600)])

    return k(h, src_e, dst_e)


NB = 2000
NBLK = N // NB
_HI = jax.lax.Precision.DEFAULT
_CN = (((1,), (1,)), ((), ()))


def _root(h, Wroot):
    def body(h_ref, wo_ref, o_ref):
        o_ref[...] = lax.dot_general(h_ref[...], wo_ref[...], _CN,
                                     precision=_HI,
                                     preferred_element_type=jnp.float32)

    return pl.pallas_call(
        body,
        grid=(NBLK,),
        in_specs=[
            pl.BlockSpec((NB, D), lambda i: (i, 0)),
            pl.BlockSpec((D, D), lambda i: (0, 0)),
        ],
        out_specs=pl.BlockSpec((NB, D), lambda i: (i, 0)),
        out_shape=jax.ShapeDtypeStruct((N, D), jnp.float32),
    )(h, Wroot)


def _dn_phase0(i, a0_ref, a1_ref, xr_ref, wr_ref, x_sc, st_sc):
    a = a0_ref[...] + a1_ref[...]
    xv = lax.dot_general(a, wr_ref[...], _CN, precision=_HI,
                         preferred_element_type=jnp.float32)
    xv = xv + xr_ref[...]
    x_sc[pl.ds(i * NB, NB), :] = xv

    @pl.when(i == 0)
    def _():
        st_sc[...] = jnp.zeros((8, D), jnp.float32)

    st_sc[0:1, :] += jnp.sum(xv, axis=0, keepdims=True)
    st_sc[1:2, :] += jnp.sum(xv * xv, axis=0, keepdims=True)

    @pl.when(i == NBLK - 1)
    def _():
        mu = st_sc[0:1, :] / N
        var = st_sc[1:2, :] / N - mu * mu
        st_sc[0:1, :] = mu
        st_sc[1:2, :] = lax.rsqrt(var + 1e-5)


def _dn_normblock(j, g_ref, be_ref, x_sc, st_sc):
    mu = st_sc[0:1, :]
    rs = st_sc[1:2, :]
    xv = x_sc[pl.ds(j * NB, NB), :]
    return jnp.maximum((xv - mu) * (rs * g_ref[...]) + be_ref[...], 0.0)


_IN_SPECS_DN = [
    pl.BlockSpec((NB, D), lambda i: (jnp.minimum(i, NBLK - 1), 0)),
    pl.BlockSpec((NB, D), lambda i: (jnp.minimum(i, NBLK - 1), 0)),
    pl.BlockSpec((NB, D), lambda i: (jnp.minimum(i, NBLK - 1), 0)),
    pl.BlockSpec((D, D), lambda i: (0, 0)),
    pl.BlockSpec((1, D), lambda i: (0, 0)),
    pl.BlockSpec((1, D), lambda i: (0, 0)),
]
_SCRATCH_DN = [
    pltpu.VMEM((N, D), jnp.float32),
    pltpu.VMEM((8, D), jnp.float32),
]


def _densenorm(parts, h, Wrel, Wroot, g, be):
    def body(a0_ref, a1_ref, xr_ref, wr_ref, g_ref, be_ref, o_ref,
             x_sc, st_sc):
        i = pl.program_id(0)

        @pl.when(i < NBLK)
        def _():
            _dn_phase0(i, a0_ref, a1_ref, xr_ref, wr_ref, x_sc, st_sc)

        @pl.when(i >= NBLK)
        def _():
            o_ref[...] = _dn_normblock(i - NBLK, g_ref, be_ref, x_sc, st_sc)

    xroot = _root(h, Wroot)
    return pl.pallas_call(
        body,
        grid=(2 * NBLK,),
        in_specs=_IN_SPECS_DN,
        out_specs=pl.BlockSpec((NB, D), lambda i: (jnp.maximum(i - NBLK, 0),
                                                   0)),
        out_shape=jax.ShapeDtypeStruct((N, D), jnp.float32),
        scratch_shapes=_SCRATCH_DN,
    )(parts[:N], parts[N:], xroot, Wrel, g.reshape(1, D), be.reshape(1, D))


def _final(parts, h, Wrel, Wroot, g, be, batch3d, W1, b1, W2, b2):
    def body(a0_ref, a1_ref, xr_ref, wr_ref, g_ref, be_ref, b3_ref,
             w1_ref, b1_ref, w2_ref, b2_ref, o_ref, x_sc, st_sc, p_sc):
        i = pl.program_id(0)

        @pl.when(i < NBLK)
        def _():
            _dn_phase0(i, a0_ref, a1_ref, xr_ref, wr_ref, x_sc, st_sc)

        @pl.when(i >= NBLK)
        def _():
            j = i - NBLK
            h4 = _dn_normblock(j, g_ref, be_ref, x_sc, st_sc)
            b = b3_ref[...].reshape(1, NB)
            onehot = (b == lax.broadcasted_iota(jnp.int32, (G, 1), 0))

            @pl.when(j == 0)
            def _():
                p_sc[...] = jnp.zeros((G, D), jnp.float32)

            p_sc[...] += lax.dot_general(
                onehot.astype(jnp.float32), h4, (((1,), (0,)), ((), ())),
                precision=_HI, preferred_element_type=jnp.float32)

        @pl.when(i == 2 * NBLK - 1)
        def _():
            x1 = jnp.maximum(
                lax.dot_general(p_sc[...], w1_ref[...], _CN, precision=_HI,
                                preferred_element_type=jnp.float32)
                + b1_ref[...], 0.0)
            o = lax.dot_general(x1, w2_ref[...], _CN, precision=_HI,
                                preferred_element_type=jnp.float32) + b2_ref[...]
            m = jnp.max(o, axis=1, keepdims=True)
            e = jnp.exp(o - m)
            lse = jnp.log(jnp.sum(e, axis=1, keepdims=True)) + m
            o_ref[...] = o - lse

    xroot = _root(h, Wroot)
    return pl.pallas_call(
        body,
        grid=(2 * NBLK,),
        in_specs=_IN_SPECS_DN + [
            pl.BlockSpec((1, 1, NB),
                         lambda i: (jnp.maximum(i - NBLK, 0), 0, 0)),
            pl.BlockSpec((D, D), lambda i: (0, 0)),
            pl.BlockSpec((1, D), lambda i: (0, 0)),
            pl.BlockSpec((C, D), lambda i: (0, 0)),
            pl.BlockSpec((1, C), lambda i: (0, 0)),
        ],
        out_specs=pl.BlockSpec((G, C), lambda i: (0, 0)),
        out_shape=jax.ShapeDtypeStruct((G, C), jnp.float32),
        scratch_shapes=_SCRATCH_DN + [pltpu.VMEM((G, D), jnp.float32)],
    )(parts[:N], parts[N:], xroot, Wrel, g.reshape(1, D),
      be.reshape(1, D), batch3d, W1, b1.reshape(1, D), W2, b2.reshape(1, C))


def kernel(x, edge_index, batch, Wrel0, Wrel1, Wrel2, Wrel3, Wroot0, Wroot1,
           Wroot2, Wroot3, bc0, bc1, bc2, bc3, g0, g1, g2, g3, be0, be1, be2,
           be3, W1, b1, W2, b2):
    src_e = edge_index[0]
    dst_e = edge_index[1]
    batch3d = batch.reshape(NBLK, 1, NB)
    Wrel = [Wrel0, Wrel1, Wrel2, Wrel3]
    Wroot = [Wroot0, Wroot1, Wroot2, Wroot3]
    gs = [g0, g1, g2, g3]
    bes = [be0, be1, be2, be3]
    h = x
    for i in range(3):
        parts = _scseg(h, src_e, dst_e)
        h = _densenorm(parts, h, Wrel[i], Wroot[i], gs[i], bes[i])
    parts = _scseg(h, src_e, dst_e)
    return _final(parts, h, Wrel[3], Wroot[3], gs[3], bes[3], batch3d,
                  W1, b1, W2, b2)


# speedup vs baseline: 1.0145x; 1.0145x over previous
import functools

import jax
import jax.numpy as jnp
from jax import lax
from jax.experimental import pallas as pl
from jax.experimental.pallas import tpu as pltpu
from jax.experimental.pallas import tpu_sc as plsc

N = 10000
E = 320000
D = 128
C = 10
G = 128

NC = 2
NS = 16
NW = NC * NS

K = 64
EPT = E // NW
CH = EPT // K
KT = EPT - CH * K
NPAD = 10240
ZR = 64
DEPTH = 4


def _scseg(h, src_e, dst_e):
    mesh = plsc.VectorSubcoreMesh(core_axis_name="c", subcore_axis_name="s")

    @functools.partial(
        pl.kernel,
        mesh=mesh,
        out_type=jax.ShapeDtypeStruct((2 * N, D), jnp.float32),
        scratch_types=(
            [pltpu.VMEM((EPT,), jnp.int32)]
            + [pltpu.VMEM((K,), jnp.int32)] * DEPTH
            + [pltpu.VMEM((KT,), jnp.int32)]
            + [pltpu.VMEM((K, D), jnp.float32)] * DEPTH
            + [pltpu.VMEM((KT, D), jnp.float32)]
            + [pltpu.VMEM_SHARED((NPAD, D), jnp.float32)]
            + [pltpu.SemaphoreType.DMA] * (2 * DEPTH + 1)
        ),
    )
    def k(h_hbm, src_hbm, dst_hbm, out_hbm, src_all, *rest):
        dst_vs = rest[0:DEPTH]
        dst_vt = rest[DEPTH]
        rows_vs = rest[DEPTH + 1:2 * DEPTH + 1]
        rows_vt = rest[2 * DEPTH + 1]
        acc_sh = rest[2 * DEPTH + 2]
        sgs = rest[2 * DEPTH + 3:3 * DEPTH + 3]
        sds = rest[3 * DEPTH + 3:4 * DEPTH + 3]
        semt = rest[4 * DEPTH + 3]
        c = lax.axis_index("c")
        s = lax.axis_index("s")

        wid = c * NS + s
        ebase = wid * EPT
        pltpu.async_copy(src_hbm.at[pl.ds(ebase, EPT)], src_all, sgs[0])

        def zrow(i, carry):
            for j in range(D // 16):
                rows_vs[0][i, pl.ds(j * 16, 16)] = jnp.zeros((16,),
                                                             jnp.float32)
            return carry

        lax.fori_loop(0, ZR, zrow, 0)

        def zcopy(i, carry):
            pltpu.sync_copy(rows_vs[0], acc_sh.at[pl.ds(s * 640 + i * ZR, ZR)])
            return carry

        lax.fori_loop(0, 640 // ZR, zcopy, 0)
        pltpu.make_async_copy(src_hbm.at[pl.ds(ebase, EPT)], src_all,
                              sgs[0]).wait()

        def gather(ch, rows, sem):
            return pltpu.async_copy(
                h_hbm.at[src_all.at[pl.ds(ch * K, K)]], rows, sem)

        def gwait(ch, rows, sem):
            pltpu.make_async_copy(
                h_hbm.at[src_all.at[pl.ds(ch * K, K)]], rows, sem).wait()

        def dstage(ch, dst_v, sem):
            pltpu.async_copy(dst_hbm.at[pl.ds(ebase + ch * K, K)], dst_v, sem)

        def dwait(ch, dst_v, sem):
            pltpu.make_async_copy(
                dst_hbm.at[pl.ds(ebase + ch * K, K)], dst_v, sem).wait()

        def scat(rows, dst_v):
            pltpu.sync_copy(rows, acc_sh.at[dst_v], add=True)

        for u in range(DEPTH):
            gather(u, rows_vs[u], sgs[u])
            dstage(u, dst_vs[u], sds[u])
        pltpu.sync_copy(dst_hbm.at[pl.ds(ebase + CH * K, KT)], dst_vt)
        pltpu.async_copy(h_hbm.at[src_all.at[pl.ds(CH * K, KT)]], rows_vt,
                         semt)
        plsc.subcore_barrier()

        pltpu.make_async_copy(h_hbm.at[src_all.at[pl.ds(CH * K, KT)]],
                              rows_vt, semt).wait()
        pltpu.sync_copy(rows_vt, acc_sh.at[dst_vt], add=True)

        def body(t, carry):
            for u in range(DEPTH):
                ch = DEPTH * t + u
                gwait(ch, rows_vs[u], sgs[u])
                dwait(ch, dst_vs[u], sds[u])
                scat(rows_vs[u], dst_vs[u])

                @pl.when(ch + DEPTH < CH)
                def _():
                    gather(ch + DEPTH, rows_vs[u], sgs[u])
                    dstage(ch + DEPTH, dst_vs[u], sds[u])

            return carry

        lax.fori_loop(0, CH // DEPTH, body, 0)
        plsc.subcore_barrier()

        @pl.when(s < NS - 1)
        def _():
            pltpu.sync_copy(acc_sh.at[pl.ds(s * 640, 640)],
                            out_hbm.at[pl.ds(c * N + s * 640, 640)])

        @pl.when(s == NS - 1)
        def _():
            pltpu.sync_copy(acc_sh.at[pl.ds(9600, N - 9600)],
                            out_hbm.at[pl.ds(c * N + 9600, N - 9600)])

    return k(h, src_e, dst_e)


NB = 2000
NBLK = N // NB
_HI = jax.lax.Precision.DEFAULT
_CN = (((1,), (1,)), ((), ()))


def _dn_phase0(i, a0_ref, a1_ref, h_ref, wr_ref, wo_ref, x_sc, st_sc):
    a = a0_ref[...] + a1_ref[...]
    xv = lax.dot_general(a, wr_ref[...], _CN, precision=_HI,
                         preferred_element_type=jnp.float32)
    xv = xv + lax.dot_general(h_ref[...], wo_ref[...], _CN, precision=_HI,
                              preferred_element_type=jnp.float32)
    x_sc[pl.ds(i * NB, NB), :] = xv

    @pl.when(i == 0)
    def _():
        st_sc[...] = jnp.zeros((8, D), jnp.float32)

    st_sc[0:1, :] += jnp.sum(xv, axis=0, keepdims=True)
    st_sc[1:2, :] += jnp.sum(xv * xv, axis=0, keepdims=True)

    @pl.when(i == NBLK - 1)
    def _():
        mu = st_sc[0:1, :] / N
        var = st_sc[1:2, :] / N - mu * mu
        st_sc[0:1, :] = mu
        st_sc[1:2, :] = lax.rsqrt(var + 1e-5)


def _dn_normblock(j, g_ref, be_ref, x_sc, st_sc):
    mu = st_sc[0:1, :]
    rs = st_sc[1:2, :]
    xv = x_sc[pl.ds(j * NB, NB), :]
    return jnp.maximum((xv - mu) * (rs * g_ref[...]) + be_ref[...], 0.0)


_IN_SPECS_DN = [
    pl.BlockSpec((NB, D), lambda i: (jnp.minimum(i, NBLK - 1), 0)),
    pl.BlockSpec((NB, D), lambda i: (jnp.minimum(i, NBLK - 1), 0)),
    pl.BlockSpec((NB, D), lambda i: (jnp.minimum(i, NBLK - 1), 0)),
    pl.BlockSpec((D, D), lambda i: (0, 0)),
    pl.BlockSpec((D, D), lambda i: (0, 0)),
    pl.BlockSpec((1, D), lambda i: (0, 0)),
    pl.BlockSpec((1, D), lambda i: (0, 0)),
]
_SCRATCH_DN = [
    pltpu.VMEM((N, D), jnp.float32),
    pltpu.VMEM((8, D), jnp.float32),
]


def _densenorm(parts, h, Wrel, Wroot, g, be):
    def body(a0_ref, a1_ref, h_ref, wr_ref, wo_ref, g_ref, be_ref, o_ref,
             x_sc, st_sc):
        i = pl.program_id(0)

        @pl.when(i < NBLK)
        def _():
            _dn_phase0(i, a0_ref, a1_ref, h_ref, wr_ref, wo_ref, x_sc, st_sc)

        @pl.when(i >= NBLK)
        def _():
            o_ref[...] = _dn_normblock(i - NBLK, g_ref, be_ref, x_sc, st_sc)

    return pl.pallas_call(
        body,
        grid=(2 * NBLK,),
        in_specs=_IN_SPECS_DN,
        out_specs=pl.BlockSpec((NB, D), lambda i: (jnp.maximum(i - NBLK, 0),
                                                   0)),
        out_shape=jax.ShapeDtypeStruct((N, D), jnp.float32),
        scratch_shapes=_SCRATCH_DN,
    )(parts[:N], parts[N:], h, Wrel, Wroot, g.reshape(1, D), be.reshape(1, D))


def _final(parts, h, Wrel, Wroot, g, be, batch3d, W1, b1, W2, b2):
    def body(a0_ref, a1_ref, h_ref, wr_ref, wo_ref, g_ref, be_ref, b3_ref,
             w1_ref, b1_ref, w2_ref, b2_ref, o_ref, x_sc, st_sc, p_sc):
        i = pl.program_id(0)

        @pl.when(i < NBLK)
        def _():
            _dn_phase0(i, a0_ref, a1_ref, h_ref, wr_ref, wo_ref, x_sc, st_sc)

        @pl.when(i >= NBLK)
        def _():
            j = i - NBLK
            h4 = _dn_normblock(j, g_ref, be_ref, x_sc, st_sc)
            b = b3_ref[...].reshape(1, NB)
            onehot = (b == lax.broadcasted_iota(jnp.int32, (G, 1), 0))

            @pl.when(j == 0)
            def _():
                p_sc[...] = jnp.zeros((G, D), jnp.float32)

            p_sc[...] += lax.dot_general(
                onehot.astype(jnp.float32), h4, (((1,), (0,)), ((), ())),
                precision=_HI, preferred_element_type=jnp.float32)

        @pl.when(i == 2 * NBLK - 1)
        def _():
            x1 = jnp.maximum(
                lax.dot_general(p_sc[...], w1_ref[...], _CN, precision=_HI,
                                preferred_element_type=jnp.float32)
                + b1_ref[...], 0.0)
            o = lax.dot_general(x1, w2_ref[...], _CN, precision=_HI,
                                preferred_element_type=jnp.float32) + b2_ref[...]
            m = jnp.max(o, axis=1, keepdims=True)
            e = jnp.exp(o - m)
            lse = jnp.log(jnp.sum(e, axis=1, keepdims=True)) + m
            o_ref[...] = o - lse

    return pl.pallas_call(
        body,
        grid=(2 * NBLK,),
        in_specs=_IN_SPECS_DN + [
            pl.BlockSpec((1, 1, NB),
                         lambda i: (jnp.maximum(i - NBLK, 0), 0, 0)),
            pl.BlockSpec((D, D), lambda i: (0, 0)),
            pl.BlockSpec((1, D), lambda i: (0, 0)),
            pl.BlockSpec((C, D), lambda i: (0, 0)),
            pl.BlockSpec((1, C), lambda i: (0, 0)),
        ],
        out_specs=pl.BlockSpec((G, C), lambda i: (0, 0)),
        out_shape=jax.ShapeDtypeStruct((G, C), jnp.float32),
        scratch_shapes=_SCRATCH_DN + [pltpu.VMEM((G, D), jnp.float32)],
    )(parts[:N], parts[N:], h, Wrel, Wroot, g.reshape(1, D),
      be.reshape(1, D), batch3d, W1, b1.reshape(1, D), W2, b2.reshape(1, C))


def kernel(x, edge_index, batch, Wrel0, Wrel1, Wrel2, Wrel3, Wroot0, Wroot1,
           Wroot2, Wroot3, bc0, bc1, bc2, bc3, g0, g1, g2, g3, be0, be1, be2,
           be3, W1, b1, W2, b2):
    src_e = edge_index[0]
    dst_e = edge_index[1]
    batch3d = batch.reshape(NBLK, 1, NB)
    Wrel = [Wrel0, Wrel1, Wrel2, Wrel3]
    Wroot = [Wroot0, Wroot1, Wroot2, Wroot3]
    gs = [g0, g1, g2, g3]
    bes = [be0, be1, be2, be3]
    h = x
    for i in range(3):
        parts = _scseg(h, src_e, dst_e)
        h = _densenorm(parts, h, Wrel[i], Wroot[i], gs[i], bes[i])
    parts = _scseg(h, src_e, dst_e)
    return _final(parts, h, Wrel[3], Wroot[3], gs[3], bes[3], batch3d,
                  W1, b1, W2, b2)


# speedup vs baseline: 1.0187x; 1.0041x over previous
import functools

import jax
import jax.numpy as jnp
from jax import lax
from jax.experimental import pallas as pl
from jax.experimental.pallas import tpu as pltpu
from jax.experimental.pallas import tpu_sc as plsc

N = 10000
E = 320000
D = 128
C = 10
G = 128

NC = 2
NS = 16
NW = NC * NS

K = 64
EPT = E // NW
CH = EPT // K
KT = EPT - CH * K
NPAD = 10240
ZR = 64
DEPTH = 4


def _scseg(h, src_e, dst_e):
    mesh = plsc.VectorSubcoreMesh(core_axis_name="c", subcore_axis_name="s")

    @functools.partial(
        pl.kernel,
        mesh=mesh,
        out_type=jax.ShapeDtypeStruct((2 * N, D), jnp.float32),
        scratch_types=(
            [pltpu.VMEM((EPT,), jnp.int32)]
            + [pltpu.VMEM((K,), jnp.int32)] * DEPTH
            + [pltpu.VMEM((KT,), jnp.int32)]
            + [pltpu.VMEM((K, D), jnp.float32)] * DEPTH
            + [pltpu.VMEM((KT, D), jnp.float32)]
            + [pltpu.VMEM_SHARED((NPAD, D), jnp.float32)]
            + [pltpu.SemaphoreType.DMA] * (2 * DEPTH + 1)
        ),
    )
    def k(h_hbm, src_hbm, dst_hbm, out_hbm, src_all, *rest):
        dst_vs = rest[0:DEPTH]
        dst_vt = rest[DEPTH]
        rows_vs = rest[DEPTH + 1:2 * DEPTH + 1]
        rows_vt = rest[2 * DEPTH + 1]
        acc_sh = rest[2 * DEPTH + 2]
        sgs = rest[2 * DEPTH + 3:3 * DEPTH + 3]
        sds = rest[3 * DEPTH + 3:4 * DEPTH + 3]
        semt = rest[4 * DEPTH + 3]
        c = lax.axis_index("c")
        s = lax.axis_index("s")

        wid = c * NS + s
        ebase = wid * EPT
        pltpu.async_copy(src_hbm.at[pl.ds(ebase, EPT)], src_all, sgs[0])

        def zrow(i, carry):
            for j in range(D // 16):
                rows_vs[0][i, pl.ds(j * 16, 16)] = jnp.zeros((16,),
                                                             jnp.float32)
            return carry

        lax.fori_loop(0, ZR, zrow, 0)

        def zcopy(i, carry):
            pltpu.async_copy(rows_vs[0],
                             acc_sh.at[pl.ds(s * 640 + i * ZR, ZR)], semt)
            return carry

        lax.fori_loop(0, 640 // ZR, zcopy, 0)

        def zdrain(i, carry):
            pltpu.make_async_copy(
                rows_vs[0], acc_sh.at[pl.ds(s * 640 + i * ZR, ZR)],
                semt).wait()
            return carry

        lax.fori_loop(0, 640 // ZR, zdrain, 0)
        pltpu.make_async_copy(src_hbm.at[pl.ds(ebase, EPT)], src_all,
                              sgs[0]).wait()

        def gather(ch, rows, sem):
            return pltpu.async_copy(
                h_hbm.at[src_all.at[pl.ds(ch * K, K)]], rows, sem)

        def gwait(ch, rows, sem):
            pltpu.make_async_copy(
                h_hbm.at[src_all.at[pl.ds(ch * K, K)]], rows, sem).wait()

        def dstage(ch, dst_v, sem):
            pltpu.async_copy(dst_hbm.at[pl.ds(ebase + ch * K, K)], dst_v, sem)

        def dwait(ch, dst_v, sem):
            pltpu.make_async_copy(
                dst_hbm.at[pl.ds(ebase + ch * K, K)], dst_v, sem).wait()

        def scat(rows, dst_v):
            pltpu.sync_copy(rows, acc_sh.at[dst_v], add=True)

        for u in range(DEPTH):
            gather(u, rows_vs[u], sgs[u])
            dstage(u, dst_vs[u], sds[u])
        pltpu.sync_copy(dst_hbm.at[pl.ds(ebase + CH * K, KT)], dst_vt)
        pltpu.async_copy(h_hbm.at[src_all.at[pl.ds(CH * K, KT)]], rows_vt,
                         semt)
        plsc.subcore_barrier()

        pltpu.make_async_copy(h_hbm.at[src_all.at[pl.ds(CH * K, KT)]],
                              rows_vt, semt).wait()
        pltpu.sync_copy(rows_vt, acc_sh.at[dst_vt], add=True)

        def body(t, carry):
            for u in range(DEPTH):
                ch = DEPTH * t + u
                gwait(ch, rows_vs[u], sgs[u])
                dwait(ch, dst_vs[u], sds[u])
                scat(rows_vs[u], dst_vs[u])

                @pl.when(ch + DEPTH < CH)
                def _():
                    gather(ch + DEPTH, rows_vs[u], sgs[u])
                    dstage(ch + DEPTH, dst_vs[u], sds[u])

            return carry

        lax.fori_loop(0, CH // DEPTH, body, 0)
        plsc.subcore_barrier()

        @pl.when(s < NS - 1)
        def _():
            pltpu.sync_copy(acc_sh.at[pl.ds(s * 640, 640)],
                            out_hbm.at[pl.ds(c * N + s * 640, 640)])

        @pl.when(s == NS - 1)
        def _():
            pltpu.sync_copy(acc_sh.at[pl.ds(9600, N - 9600)],
                            out_hbm.at[pl.ds(c * N + 9600, N - 9600)])

    return k(h, src_e, dst_e)


NB = 2000
NBLK = N // NB
_HI = jax.lax.Precision.DEFAULT
_CN = (((1,), (1,)), ((), ()))


def _dn_phase0(i, a0_ref, a1_ref, h_ref, wr_ref, wo_ref, x_sc, st_sc):
    a = a0_ref[...] + a1_ref[...]
    xv = lax.dot_general(a, wr_ref[...], _CN, precision=_HI,
                         preferred_element_type=jnp.float32)
    xv = xv + lax.dot_general(h_ref[...], wo_ref[...], _CN, precision=_HI,
                              preferred_element_type=jnp.float32)
    x_sc[pl.ds(i * NB, NB), :] = xv

    @pl.when(i == 0)
    def _():
        st_sc[...] = jnp.zeros((8, D), jnp.float32)

    st_sc[0:1, :] += jnp.sum(xv, axis=0, keepdims=True)
    st_sc[1:2, :] += jnp.sum(xv * xv, axis=0, keepdims=True)

    @pl.when(i == NBLK - 1)
    def _():
        mu = st_sc[0:1, :] / N
        var = st_sc[1:2, :] / N - mu * mu
        st_sc[0:1, :] = mu
        st_sc[1:2, :] = lax.rsqrt(var + 1e-5)


def _dn_normblock(j, g_ref, be_ref, x_sc, st_sc):
    mu = st_sc[0:1, :]
    rs = st_sc[1:2, :]
    xv = x_sc[pl.ds(j * NB, NB), :]
    return jnp.maximum((xv - mu) * (rs * g_ref[...]) + be_ref[...], 0.0)


_IN_SPECS_DN = [
    pl.BlockSpec((NB, D), lambda i: (jnp.minimum(i, NBLK - 1), 0)),
    pl.BlockSpec((NB, D), lambda i: (jnp.minimum(i, NBLK - 1), 0)),
    pl.BlockSpec((NB, D), lambda i: (jnp.minimum(i, NBLK - 1), 0)),
    pl.BlockSpec((D, D), lambda i: (0, 0)),
    pl.BlockSpec((D, D), lambda i: (0, 0)),
    pl.BlockSpec((1, D), lambda i: (0, 0)),
    pl.BlockSpec((1, D), lambda i: (0, 0)),
]
_SCRATCH_DN = [
    pltpu.VMEM((N, D), jnp.float32),
    pltpu.VMEM((8, D), jnp.float32),
]


def _densenorm(parts, h, Wrel, Wroot, g, be):
    def body(a0_ref, a1_ref, h_ref, wr_ref, wo_ref, g_ref, be_ref, o_ref,
             x_sc, st_sc):
        i = pl.program_id(0)

        @pl.when(i < NBLK)
        def _():
            _dn_phase0(i, a0_ref, a1_ref, h_ref, wr_ref, wo_ref, x_sc, st_sc)

        @pl.when(i >= NBLK)
        def _():
            o_ref[...] = _dn_normblock(i - NBLK, g_ref, be_ref, x_sc, st_sc)

    return pl.pallas_call(
        body,
        grid=(2 * NBLK,),
        in_specs=_IN_SPECS_DN,
        out_specs=pl.BlockSpec((NB, D), lambda i: (jnp.maximum(i - NBLK, 0),
                                                   0)),
        out_shape=jax.ShapeDtypeStruct((N, D), jnp.float32),
        scratch_shapes=_SCRATCH_DN,
    )(parts[:N], parts[N:], h, Wrel, Wroot, g.reshape(1, D), be.reshape(1, D))


def _final(parts, h, Wrel, Wroot, g, be, batch3d, W1, b1, W2, b2):
    def body(a0_ref, a1_ref, h_ref, wr_ref, wo_ref, g_ref, be_ref, b3_ref,
             w1_ref, b1_ref, w2_ref, b2_ref, o_ref, x_sc, st_sc, p_sc):
        i = pl.program_id(0)

        @pl.when(i < NBLK)
        def _():
            _dn_phase0(i, a0_ref, a1_ref, h_ref, wr_ref, wo_ref, x_sc, st_sc)

        @pl.when(i >= NBLK)
        def _():
            j = i - NBLK
            h4 = _dn_normblock(j, g_ref, be_ref, x_sc, st_sc)
            b = b3_ref[...].reshape(1, NB)
            onehot = (b == lax.broadcasted_iota(jnp.int32, (G, 1), 0))

            @pl.when(j == 0)
            def _():
                p_sc[...] = jnp.zeros((G, D), jnp.float32)

            p_sc[...] += lax.dot_general(
                onehot.astype(jnp.float32), h4, (((1,), (0,)), ((), ())),
                precision=_HI, preferred_element_type=jnp.float32)

        @pl.when(i == 2 * NBLK - 1)
        def _():
            x1 = jnp.maximum(
                lax.dot_general(p_sc[...], w1_ref[...], _CN, precision=_HI,
                                preferred_element_type=jnp.float32)
                + b1_ref[...], 0.0)
            o = lax.dot_general(x1, w2_ref[...], _CN, precision=_HI,
                                preferred_element_type=jnp.float32) + b2_ref[...]
            m = jnp.max(o, axis=1, keepdims=True)
            e = jnp.exp(o - m)
            lse = jnp.log(jnp.sum(e, axis=1, keepdims=True)) + m
            o_ref[...] = o - lse

    return pl.pallas_call(
        body,
        grid=(2 * NBLK,),
        in_specs=_IN_SPECS_DN + [
            pl.BlockSpec((1, 1, NB),
                         lambda i: (jnp.maximum(i - NBLK, 0), 0, 0)),
            pl.BlockSpec((D, D), lambda i: (0, 0)),
            pl.BlockSpec((1, D), lambda i: (0, 0)),
            pl.BlockSpec((C, D), lambda i: (0, 0)),
            pl.BlockSpec((1, C), lambda i: (0, 0)),
        ],
        out_specs=pl.BlockSpec((G, C), lambda i: (0, 0)),
        out_shape=jax.ShapeDtypeStruct((G, C), jnp.float32),
        scratch_shapes=_SCRATCH_DN + [pltpu.VMEM((G, D), jnp.float32)],
    )(parts[:N], parts[N:], h, Wrel, Wroot, g.reshape(1, D),
      be.reshape(1, D), batch3d, W1, b1.reshape(1, D), W2, b2.reshape(1, C))


def kernel(x, edge_index, batch, Wrel0, Wrel1, Wrel2, Wrel3, Wroot0, Wroot1,
           Wroot2, Wroot3, bc0, bc1, bc2, bc3, g0, g1, g2, g3, be0, be1, be2,
           be3, W1, b1, W2, b2):
    src_e = edge_index[0]
    dst_e = edge_index[1]
    batch3d = batch.reshape(NBLK, 1, NB)
    Wrel = [Wrel0, Wrel1, Wrel2, Wrel3]
    Wroot = [Wroot0, Wroot1, Wroot2, Wroot3]
    gs = [g0, g1, g2, g3]
    bes = [be0, be1, be2, be3]
    h = x
    for i in range(3):
        parts = _scseg(h, src_e, dst_e)
        h = _densenorm(parts, h, Wrel[i], Wroot[i], gs[i], bes[i])
    parts = _scseg(h, src_e, dst_e)
    return _final(parts, h, Wrel[3], Wroot[3], gs[3], bes[3], batch3d,
                  W1, b1, W2, b2)


# speedup vs baseline: 1.0206x; 1.0018x over previous
import functools

import jax
import jax.numpy as jnp
from jax import lax
from jax.experimental import pallas as pl
from jax.experimental.pallas import tpu as pltpu
from jax.experimental.pallas import tpu_sc as plsc

N = 10000
E = 320000
D = 128
C = 10
G = 128

NC = 2
NS = 16
NW = NC * NS

K = 48
EPT = E // NW
CH = EPT // K
KT = EPT - CH * K
NPAD = 10240
ZR = 48
DEPTH = 6


def _scseg(h, src_e, dst_e):
    mesh = plsc.VectorSubcoreMesh(core_axis_name="c", subcore_axis_name="s")

    @functools.partial(
        pl.kernel,
        mesh=mesh,
        out_type=jax.ShapeDtypeStruct((2 * N, D), jnp.float32),
        scratch_types=(
            [pltpu.VMEM((EPT,), jnp.int32)]
            + [pltpu.VMEM((K,), jnp.int32)] * DEPTH
            + [pltpu.VMEM((KT,), jnp.int32)]
            + [pltpu.VMEM((K, D), jnp.float32)] * DEPTH
            + [pltpu.VMEM_SHARED((NPAD, D), jnp.float32)]
            + [pltpu.SemaphoreType.DMA] * (2 * DEPTH + 1)
        ),
    )
    def k(h_hbm, src_hbm, dst_hbm, out_hbm, src_all, *rest):
        dst_vs = rest[0:DEPTH]
        dst_vt = rest[DEPTH]
        rows_vs = rest[DEPTH + 1:2 * DEPTH + 1]
        acc_sh = rest[2 * DEPTH + 1]
        sgs = rest[2 * DEPTH + 2:3 * DEPTH + 2]
        sds = rest[3 * DEPTH + 2:4 * DEPTH + 2]
        semt = rest[4 * DEPTH + 2]
        c = lax.axis_index("c")
        s = lax.axis_index("s")

        wid = c * NS + s
        ebase = wid * EPT
        pltpu.async_copy(src_hbm.at[pl.ds(ebase, EPT)], src_all, sgs[0])

        def zrow(i, carry):
            for j in range(D // 16):
                rows_vs[0][i, pl.ds(j * 16, 16)] = jnp.zeros((16,),
                                                             jnp.float32)
            return carry

        lax.fori_loop(0, ZR, zrow, 0)

        def zcopy(i, carry):
            pltpu.async_copy(rows_vs[0],
                             acc_sh.at[pl.ds(s * 640 + i * ZR, ZR)], semt)
            return carry

        lax.fori_loop(0, 640 // ZR, zcopy, 0)
        pltpu.async_copy(rows_vs[0].at[pl.ds(0, 16)],
                         acc_sh.at[pl.ds(s * 640 + (640 // ZR) * ZR, 16)],
                         semt)

        def zdrain(i, carry):
            pltpu.make_async_copy(
                rows_vs[0], acc_sh.at[pl.ds(s * 640 + i * ZR, ZR)],
                semt).wait()
            return carry

        lax.fori_loop(0, 640 // ZR, zdrain, 0)
        pltpu.make_async_copy(rows_vs[0].at[pl.ds(0, 16)],
                              acc_sh.at[pl.ds(s * 640 + (640 // ZR) * ZR, 16)],
                              semt).wait()
        pltpu.make_async_copy(src_hbm.at[pl.ds(ebase, EPT)], src_all,
                              sgs[0]).wait()

        def gather(ch, rows, sem):
            return pltpu.async_copy(
                h_hbm.at[src_all.at[pl.ds(ch * K, K)]], rows, sem)

        def gwait(ch, rows, sem):
            pltpu.make_async_copy(
                h_hbm.at[src_all.at[pl.ds(ch * K, K)]], rows, sem).wait()

        def dstage(ch, dst_v, sem):
            pltpu.async_copy(dst_hbm.at[pl.ds(ebase + ch * K, K)], dst_v, sem)

        def dwait(ch, dst_v, sem):
            pltpu.make_async_copy(
                dst_hbm.at[pl.ds(ebase + ch * K, K)], dst_v, sem).wait()

        def scat(rows, dst_v):
            pltpu.sync_copy(rows, acc_sh.at[dst_v], add=True)

        pltpu.sync_copy(dst_hbm.at[pl.ds(ebase + CH * K, KT)], dst_vt)
        pltpu.async_copy(h_hbm.at[src_all.at[pl.ds(CH * K, KT)]],
                         rows_vs[0].at[pl.ds(0, KT)], semt)
        for u in range(1, DEPTH):
            gather(u, rows_vs[u], sgs[u])
            dstage(u, dst_vs[u], sds[u])
        dstage(0, dst_vs[0], sds[0])
        plsc.subcore_barrier()

        pltpu.make_async_copy(h_hbm.at[src_all.at[pl.ds(CH * K, KT)]],
                              rows_vs[0].at[pl.ds(0, KT)], semt).wait()
        pltpu.sync_copy(rows_vs[0].at[pl.ds(0, KT)], acc_sh.at[dst_vt],
                        add=True)
        gather(0, rows_vs[0], sgs[0])

        def body(t, carry):
            for u in range(DEPTH):
                ch = DEPTH * t + u

                @pl.when(ch < CH)
                def _():
                    gwait(ch, rows_vs[u], sgs[u])
                    dwait(ch, dst_vs[u], sds[u])
                    scat(rows_vs[u], dst_vs[u])

                @pl.when(ch + DEPTH < CH)
                def _():
                    gather(ch + DEPTH, rows_vs[u], sgs[u])
                    dstage(ch + DEPTH, dst_vs[u], sds[u])

            return carry

        lax.fori_loop(0, (CH + DEPTH - 1) // DEPTH, body, 0)
        plsc.subcore_barrier()

        @pl.when(s < NS - 1)
        def _():
            pltpu.sync_copy(acc_sh.at[pl.ds(s * 640, 640)],
                            out_hbm.at[pl.ds(c * N + s * 640, 640)])

        @pl.when(s == NS - 1)
        def _():
            pltpu.sync_copy(acc_sh.at[pl.ds(9600, N - 9600)],
                            out_hbm.at[pl.ds(c * N + 9600, N - 9600)])

    return k(h, src_e, dst_e)


NB = 2000
NBLK = N // NB
_HI = jax.lax.Precision.DEFAULT
_CN = (((1,), (1,)), ((), ()))


def _dn_phase0(i, a0_ref, a1_ref, h_ref, wr_ref, wo_ref, x_sc, st_sc):
    a = a0_ref[...] + a1_ref[...]
    xv = lax.dot_general(a, wr_ref[...], _CN, precision=_HI,
                         preferred_element_type=jnp.float32)
    xv = xv + lax.dot_general(h_ref[...], wo_ref[...], _CN, precision=_HI,
                              preferred_element_type=jnp.float32)
    x_sc[pl.ds(i * NB, NB), :] = xv

    @pl.when(i == 0)
    def _():
        st_sc[...] = jnp.zeros((8, D), jnp.float32)

    st_sc[0:1, :] += jnp.sum(xv, axis=0, keepdims=True)
    st_sc[1:2, :] += jnp.sum(xv * xv, axis=0, keepdims=True)

    @pl.when(i == NBLK - 1)
    def _():
        mu = st_sc[0:1, :] / N
        var = st_sc[1:2, :] / N - mu * mu
        st_sc[0:1, :] = mu
        st_sc[1:2, :] = lax.rsqrt(var + 1e-5)


def _dn_normblock(j, g_ref, be_ref, x_sc, st_sc):
    mu = st_sc[0:1, :]
    rs = st_sc[1:2, :]
    xv = x_sc[pl.ds(j * NB, NB), :]
    return jnp.maximum((xv - mu) * (rs * g_ref[...]) + be_ref[...], 0.0)


_IN_SPECS_DN = [
    pl.BlockSpec((NB, D), lambda i: (jnp.minimum(i, NBLK - 1), 0)),
    pl.BlockSpec((NB, D), lambda i: (jnp.minimum(i, NBLK - 1), 0)),
    pl.BlockSpec((NB, D), lambda i: (jnp.minimum(i, NBLK - 1), 0)),
    pl.BlockSpec((D, D), lambda i: (0, 0)),
    pl.BlockSpec((D, D), lambda i: (0, 0)),
    pl.BlockSpec((1, D), lambda i: (0, 0)),
    pl.BlockSpec((1, D), lambda i: (0, 0)),
]
_SCRATCH_DN = [
    pltpu.VMEM((N, D), jnp.float32),
    pltpu.VMEM((8, D), jnp.float32),
]


def _densenorm(parts, h, Wrel, Wroot, g, be):
    def body(a0_ref, a1_ref, h_ref, wr_ref, wo_ref, g_ref, be_ref, o_ref,
             x_sc, st_sc):
        i = pl.program_id(0)

        @pl.when(i < NBLK)
        def _():
            _dn_phase0(i, a0_ref, a1_ref, h_ref, wr_ref, wo_ref, x_sc, st_sc)

        @pl.when(i >= NBLK)
        def _():
            o_ref[...] = _dn_normblock(i - NBLK, g_ref, be_ref, x_sc, st_sc)

    return pl.pallas_call(
        body,
        grid=(2 * NBLK,),
        in_specs=_IN_SPECS_DN,
        out_specs=pl.BlockSpec((NB, D), lambda i: (jnp.maximum(i - NBLK, 0),
                                                   0)),
        out_shape=jax.ShapeDtypeStruct((N, D), jnp.float32),
        scratch_shapes=_SCRATCH_DN,
    )(parts[:N], parts[N:], h, Wrel, Wroot, g.reshape(1, D), be.reshape(1, D))


def _final(parts, h, Wrel, Wroot, g, be, batch3d, W1, b1, W2, b2):
    def body(a0_ref, a1_ref, h_ref, wr_ref, wo_ref, g_ref, be_ref, b3_ref,
             w1_ref, b1_ref, w2_ref, b2_ref, o_ref, x_sc, st_sc, p_sc):
        i = pl.program_id(0)

        @pl.when(i < NBLK)
        def _():
            _dn_phase0(i, a0_ref, a1_ref, h_ref, wr_ref, wo_ref, x_sc, st_sc)

        @pl.when(i >= NBLK)
        def _():
            j = i - NBLK
            h4 = _dn_normblock(j, g_ref, be_ref, x_sc, st_sc)
            b = b3_ref[...].reshape(1, NB)
            onehot = (b == lax.broadcasted_iota(jnp.int32, (G, 1), 0))

            @pl.when(j == 0)
            def _():
                p_sc[...] = jnp.zeros((G, D), jnp.float32)

            p_sc[...] += lax.dot_general(
                onehot.astype(jnp.float32), h4, (((1,), (0,)), ((), ())),
                precision=_HI, preferred_element_type=jnp.float32)

        @pl.when(i == 2 * NBLK - 1)
        def _():
            x1 = jnp.maximum(
                lax.dot_general(p_sc[...], w1_ref[...], _CN, precision=_HI,
                                preferred_element_type=jnp.float32)
                + b1_ref[...], 0.0)
            o = lax.dot_general(x1, w2_ref[...], _CN, precision=_HI,
                                preferred_element_type=jnp.float32) + b2_ref[...]
            m = jnp.max(o, axis=1, keepdims=True)
            e = jnp.exp(o - m)
            lse = jnp.log(jnp.sum(e, axis=1, keepdims=True)) + m
            o_ref[...] = o - lse

    return pl.pallas_call(
        body,
        grid=(2 * NBLK,),
        in_specs=_IN_SPECS_DN + [
            pl.BlockSpec((1, 1, NB),
                         lambda i: (jnp.maximum(i - NBLK, 0), 0, 0)),
            pl.BlockSpec((D, D), lambda i: (0, 0)),
            pl.BlockSpec((1, D), lambda i: (0, 0)),
            pl.BlockSpec((C, D), lambda i: (0, 0)),
            pl.BlockSpec((1, C), lambda i: (0, 0)),
        ],
        out_specs=pl.BlockSpec((G, C), lambda i: (0, 0)),
        out_shape=jax.ShapeDtypeStruct((G, C), jnp.float32),
        scratch_shapes=_SCRATCH_DN + [pltpu.VMEM((G, D), jnp.float32)],
    )(parts[:N], parts[N:], h, Wrel, Wroot, g.reshape(1, D),
      be.reshape(1, D), batch3d, W1, b1.reshape(1, D), W2, b2.reshape(1, C))


def kernel(x, edge_index, batch, Wrel0, Wrel1, Wrel2, Wrel3, Wroot0, Wroot1,
           Wroot2, Wroot3, bc0, bc1, bc2, bc3, g0, g1, g2, g3, be0, be1, be2,
           be3, W1, b1, W2, b2):
    src_e = edge_index[0]
    dst_e = edge_index[1]
    batch3d = batch.reshape(NBLK, 1, NB)
    Wrel = [Wrel0, Wrel1, Wrel2, Wrel3]
    Wroot = [Wroot0, Wroot1, Wroot2, Wroot3]
    gs = [g0, g1, g2, g3]
    bes = [be0, be1, be2, be3]
    h = x
    for i in range(3):
        parts = _scseg(h, src_e, dst_e)
        h = _densenorm(parts, h, Wrel[i], Wroot[i], gs[i], bes[i])
    parts = _scseg(h, src_e, dst_e)
    return _final(parts, h, Wrel[3], Wroot[3], gs[3], bes[3], batch3d,
                  W1, b1, W2, b2)
